# baseline (speedup 1.0000x reference)
import functools

import jax
import jax.numpy as jnp
from jax import lax
from jax.experimental import pallas as pl
from jax.experimental.pallas import tpu as pltpu
from jax.experimental.pallas import tpu_sc as plsc

_N = 10000
_D = 128
_N_PAD = 10240
_CH = 64
_NSUB = 16
_NWORK = 32
_KCH = 158
_KCH0 = 124
_KCH1 = 2 * _KCH - _KCH0
_E_PAD = _NWORK * _KCH * _CH
_E_ALLOC = _E_PAD + 2 * _CH
_ROWS_PER_TILE = _N_PAD // _NSUB
_SLOPE = 0.2


_NDEN = _N_PAD // _D


def _edge_body(xl_hbm, xr_hbm, src_hbm, dst_hbm, a_hbm, msg_hbm, den_hbm,
               srcbuf, dstbuf, dscat, glbuf, grbuf, dloc, idxiota, wtmp, abuf,
               sacc, sden, semg0, semg1, semi0, semi1, sems0, sems1):
    c = lax.axis_index("c")
    s = lax.axis_index("s")
    wid = c * _NSUB + s
    semg = (semg0, semg1)
    semi = (semi0, semi1)
    sems = (sems0, sems1)

    pltpu.sync_copy(a_hbm, abuf)

    lanes = lax.iota(jnp.int32, 16)
    zero16 = jnp.zeros((16,), jnp.float32)

    def _zrow(i, carry):
        for j in range(_D // 16):
            glbuf[0, i, pl.ds(j * 16, 16)] = zero16
        return carry

    lax.fori_loop(0, _CH, _zrow, 0)

    def _zden(i, carry):
        for j in range(_D // 16):
            dloc[i, pl.ds(j * 16, 16)] = zero16
        return carry

    lax.fori_loop(0, _NDEN, _zden, 0)

    def _ziota(i, carry):
        idxiota[pl.ds(i * 16, 16)] = i * 16 + lanes
        return carry

    lax.fori_loop(0, _NDEN // 16, _ziota, 0)

    for k in range(_ROWS_PER_TILE // _CH):
        pltpu.sync_copy(glbuf.at[0],
                        sacc.at[pl.ds(s * _ROWS_PER_TILE + k * _CH, _CH)])

    @pl.when(s == 0)
    def _init_sden():
        pltpu.sync_copy(glbuf.at[0], sden.at[pl.ds(0, _CH)])
        pltpu.sync_copy(glbuf.at[0, pl.ds(0, _NDEN - _CH)],
                        sden.at[pl.ds(_CH, _NDEN - _CH)])

    plsc.subcore_barrier()

    av = [abuf[pl.ds(jb * 16, 16)] for jb in range(_D // 16)]
    lane0 = lanes == 0
    lanesm1 = jnp.maximum(lanes - 1, 0)
    lanesp1 = jnp.minimum(lanes + 1, 15)
    ebase = jnp.where(c == 0, s * _KCH0,
                      _NSUB * _KCH0 + s * _KCH1) * _CH
    nch = jnp.where(c == 0, _KCH0, _KCH1)

    def _idx_issue(slot, k):
        base = ebase + k * _CH
        pltpu.async_copy(src_hbm.at[pl.ds(base, _CH)], srcbuf.at[slot],
                         semi[slot])
        pltpu.async_copy(dst_hbm.at[pl.ds(base, _CH)], dstbuf.at[slot],
                         semi[slot])

    def _idx_drain(slot):
        pltpu.make_async_copy(src_hbm.at[pl.ds(0, _CH)], srcbuf.at[slot],
                              semi[slot]).wait()
        pltpu.make_async_copy(dst_hbm.at[pl.ds(0, _CH)], dstbuf.at[slot],
                              semi[slot]).wait()

    def _gather_issue(slot):
        pltpu.async_copy(xl_hbm.at[srcbuf.at[slot]], glbuf.at[slot],
                         semg[slot])
        pltpu.async_copy(xr_hbm.at[dstbuf.at[slot]], grbuf.at[slot],
                         semg[slot])

    def _gather_drain(slot):
        pltpu.make_async_copy(xl_hbm.at[srcbuf.at[slot]], glbuf.at[slot],
                              semg[slot]).wait()
        pltpu.make_async_copy(xr_hbm.at[dstbuf.at[slot]], grbuf.at[slot],
                              semg[slot]).wait()

    def _msg_drain(slot):
        pltpu.make_async_copy(grbuf.at[slot], sacc.at[dscat.at[slot]],
                              sems[slot]).wait()

    def _chunk_step(k, sl, other, first=False):
        _idx_drain(other)
        if not first:
            _msg_drain(other)
        _gather_issue(other)
        _gather_drain(sl)

        @plsc.parallel_loop(0, _CH, unroll=2)
        def _edge(e):
            acc = jnp.zeros((16,), jnp.float32)
            gls = []
            for jb in range(_D // 16):
                gl = glbuf[sl, e, pl.ds(jb * 16, 16)]
                gr = grbuf[sl, e, pl.ds(jb * 16, 16)]
                gls.append(gl)
                m = gl + gr
                lr = jnp.maximum(m, _SLOPE * m)
                acc = acc + av[jb] * lr
            logit = jnp.sum(acc)
            w = jnp.exp(jnp.full((16,), logit, jnp.float32))
            for jb in range(_D // 16):
                grbuf[sl, e, pl.ds(jb * 16, 16)] = gls[jb] * w
            plsc.store_scatter(wtmp, [jnp.full((16,), e, jnp.int32)], w,
                               mask=lane0)

        @plsc.parallel_loop(0, _CH // 16, unroll=2)
        def _dcopy(g):
            dscat[sl, pl.ds(g * 16, 16)] = dstbuf[sl, pl.ds(g * 16, 16)]

        pltpu.async_copy(grbuf.at[sl], sacc.at[dscat.at[sl]], sems[sl],
                         add=True)

        @plsc.parallel_loop(0, _CH // 16, unroll=2)
        def _dgroup(g):
            dstv = dscat[sl, pl.ds(g * 16, 16)]
            w16 = wtmp[pl.ds(g * 16, 16)]
            ks, vs = plsc.sort_key_val(dstv, w16)
            ksh = ks.at[lanesm1].get(mode="promise_in_bounds")
            b = (lanes == 0) | (ks != ksh)
            st = plsc.cummax(jnp.where(b, lanes, 0))
            cs = plsc.cumsum(vs)
            csm = cs.at[jnp.maximum(st - 1, 0)].get(mode="promise_in_bounds")
            seg = cs - jnp.where(st == 0, 0.0, csm)
            ksn = ks.at[lanesp1].get(mode="promise_in_bounds")
            end = (lanes == 15) | (ks != ksn)
            rowi = lax.shift_right_logical(ks, 7)
            coli = lax.bitwise_and(ks, _D - 1)
            plsc.addupdate_scatter(dloc, [rowi, coli], seg, mask=end)

        _idx_issue(sl, k + 2)

    pltpu.sync_copy(src_hbm.at[pl.ds(ebase, _CH)], srcbuf.at[0])
    pltpu.sync_copy(dst_hbm.at[pl.ds(ebase, _CH)], dstbuf.at[0])
    _gather_issue(0)
    _idx_issue(1, 1)

    _chunk_step(0, 0, 1, first=True)
    _chunk_step(1, 1, 0)

    def _pair(kk, carry):
        k = kk * 2
        _chunk_step(k, 0, 1)
        _chunk_step(k + 1, 1, 0)
        return carry

    lax.fori_loop(1, nch // 2, _pair, 0)

    _gather_drain(0)
    _idx_drain(1)
    _msg_drain(1)

    pltpu.sync_copy(dloc, sden.at[idxiota], add=True)

    plsc.subcore_barrier()
    rb = s * _ROWS_PER_TILE
    pltpu.sync_copy(sacc.at[pl.ds(rb, _ROWS_PER_TILE)],
                    msg_hbm.at[c, pl.ds(rb, _ROWS_PER_TILE)])

    @pl.when(s < 10)
    def _den_out():
        pltpu.sync_copy(sden.at[pl.ds(s * 8, 8)],
                        den_hbm.at[c, pl.ds(s * 8, 8)])


def _edge_call(xl, xr, srcp, dstp, a):
    mesh = plsc.VectorSubcoreMesh(core_axis_name="c", subcore_axis_name="s")
    fn = pl.kernel(
        _edge_body,
        mesh=mesh,
        compiler_params=pltpu.CompilerParams(needs_layout_passes=False),
        out_type=[
            jax.ShapeDtypeStruct((2, _N_PAD, _D), jnp.float32),
            jax.ShapeDtypeStruct((2, _NDEN, _D), jnp.float32),
        ],
        scratch_types=[
            pltpu.VMEM((2, _CH), jnp.int32),
            pltpu.VMEM((2, _CH), jnp.int32),
            pltpu.VMEM((2, _CH), jnp.int32),
            pltpu.VMEM((2, _CH, _D), jnp.float32),
            pltpu.VMEM((2, _CH, _D), jnp.float32),
            pltpu.VMEM((_NDEN, _D), jnp.float32),
            pltpu.VMEM((_NDEN,), jnp.int32),
            pltpu.VMEM((_CH,), jnp.float32),
            pltpu.VMEM((_D,), jnp.float32),
            pltpu.VMEM_SHARED((_N_PAD, _D), jnp.float32),
            pltpu.VMEM_SHARED((_NDEN, _D), jnp.float32),
            pltpu.SemaphoreType.DMA,
            pltpu.SemaphoreType.DMA,
            pltpu.SemaphoreType.DMA,
            pltpu.SemaphoreType.DMA,
            pltpu.SemaphoreType.DMA,
            pltpu.SemaphoreType.DMA,
        ],
    )
    return fn(xl, xr, srcp, dstp, a)


_ROWS_BLK = 1024


def _prologue_body(x_ref, wl_ref, bl_ref, wr_ref, br_ref, xl_ref, xr_ref):
    x = x_ref[...]
    xl_ref[...] = jnp.dot(x, wl_ref[...], preferred_element_type=jnp.float32) + bl_ref[...]
    xr_ref[...] = jnp.dot(x, wr_ref[...], preferred_element_type=jnp.float32) + br_ref[...]


def _combine_h(xl, xr, am0, am1, ad0, ad1, a_row, bias_row):
    ssum = xl + xr
    lr = jnp.maximum(ssum, _SLOPE * ssum)
    cself = jnp.sum(lr * a_row, axis=1, keepdims=True)
    scv = jnp.exp(cself)
    num = scv * xl + am0 + am1
    den = scv + ad0[:, 0:1] + ad1[:, 0:1] + 1e-16
    return jnp.maximum(num / den + bias_row, 0.0)


def _combine_prologue_body(xl_ref, xr_ref, am0_ref, am1_ref, ad0_ref, ad1_ref,
                           a_ref, bias_ref, wl_ref, bl_ref, wr_ref, br_ref,
                           ol_ref, or_ref):
    h = _combine_h(xl_ref[...], xr_ref[...], am0_ref[...], am1_ref[...],
                   ad0_ref[...], ad1_ref[...], a_ref[...], bias_ref[...])
    ol_ref[...] = jnp.dot(h, wl_ref[...], preferred_element_type=jnp.float32) + bl_ref[...]
    or_ref[...] = jnp.dot(h, wr_ref[...], preferred_element_type=jnp.float32) + br_ref[...]


def _final_body(xl_ref, xr_ref, am0_ref, am1_ref, ad0_ref, ad1_ref,
                a_ref, bias_ref, wo_ref, bo_ref, out_ref):
    h = _combine_h(xl_ref[...], xr_ref[...], am0_ref[...], am1_ref[...],
                   ad0_ref[...], ad1_ref[...], a_ref[...], bias_ref[...])
    z = jnp.dot(h, wo_ref[...], preferred_element_type=jnp.float32) + bo_ref[...]
    m = jnp.max(z, axis=1, keepdims=True)
    ez = jnp.exp(z - m)
    out_ref[...] = (z - m) - jnp.log(jnp.sum(ez, axis=1, keepdims=True))


def _row_spec():
    return pl.BlockSpec((_ROWS_BLK, _D), lambda i: (i, 0))


def _den_spec():
    return pl.BlockSpec((_ROWS_BLK, 16), lambda i: (i, 0))


def _w_spec():
    return pl.BlockSpec((_D, _D), lambda i: (0, 0))


def _b_spec():
    return pl.BlockSpec((1, _D), lambda i: (0, 0))


def _tc_prologue(x, wl, bl, wr, br):
    grid = (_N_PAD // _ROWS_BLK,)
    return pl.pallas_call(
        _prologue_body,
        grid=grid,
        in_specs=[_row_spec(), _w_spec(), _b_spec(), _w_spec(), _b_spec()],
        out_specs=[_row_spec(), _row_spec()],
        out_shape=[jax.ShapeDtypeStruct((_N_PAD, _D), jnp.float32)] * 2,
    )(x, wl, bl.reshape(1, _D), wr, br.reshape(1, _D))


def _split_acc(msg, den):
    denv = den.reshape(2, _N_PAD)
    ad0 = jnp.broadcast_to(denv[0][:, None], (_N_PAD, 16))
    ad1 = jnp.broadcast_to(denv[1][:, None], (_N_PAD, 16))
    return msg[0], msg[1], ad0, ad1


def _tc_combine_prologue(xl, xr, acc, a, bias, wl, bl, wr, br):
    grid = (_N_PAD // _ROWS_BLK,)
    am0, am1, ad0, ad1 = _split_acc(*acc)
    return pl.pallas_call(
        _combine_prologue_body,
        grid=grid,
        in_specs=[_row_spec(), _row_spec(), _row_spec(), _row_spec(),
                  _den_spec(), _den_spec(), _b_spec(), _b_spec(),
                  _w_spec(), _b_spec(), _w_spec(), _b_spec()],
        out_specs=[_row_spec(), _row_spec()],
        out_shape=[jax.ShapeDtypeStruct((_N_PAD, _D), jnp.float32)] * 2,
    )(xl, xr, am0, am1, ad0, ad1, a.reshape(1, _D), bias.reshape(1, _D),
      wl, bl.reshape(1, _D), wr, br.reshape(1, _D))


def _tc_final(xl, xr, acc, a, bias, wo, bo):
    grid = (_N_PAD // _ROWS_BLK,)
    am0, am1, ad0, ad1 = _split_acc(*acc)
    return pl.pallas_call(
        _final_body,
        grid=grid,
        in_specs=[_row_spec(), _row_spec(), _row_spec(), _row_spec(),
                  _den_spec(), _den_spec(), _b_spec(), _b_spec(),
                  _w_spec(), _b_spec()],
        out_specs=_row_spec(),
        out_shape=jax.ShapeDtypeStruct((_N_PAD, _D), jnp.float32),
    )(xl, xr, am0, am1, ad0, ad1, a.reshape(1, _D), bias.reshape(1, _D),
      wo, bo.reshape(1, _D))


def kernel(x, edge_index, W1l, b1l, W1r, b1r, a1, c1, W2l, b2l, W2r, b2r,
           a2, c2, Wo, bo):
    xp = jnp.pad(x, ((0, _N_PAD - _N), (0, 0)))
    src = edge_index[0].astype(jnp.int32)
    dst = edge_index[1].astype(jnp.int32)
    pad_e = _E_ALLOC - src.shape[0]
    srcp = jnp.concatenate([src, jnp.zeros((pad_e,), jnp.int32)])
    dstp = jnp.concatenate([dst, jnp.full((pad_e,), _N, jnp.int32)])

    xl1, xr1 = _tc_prologue(xp, W1l, b1l, W1r, b1r)
    acc1 = _edge_call(xl1, xr1, srcp, dstp, a1)
    xl2, xr2 = _tc_combine_prologue(xl1, xr1, acc1, a1, c1, W2l, b2l, W2r, b2r)
    acc2 = _edge_call(xl2, xr2, srcp, dstp, a2)
    outp = _tc_final(xl2, xr2, acc2, a2, c2, Wo, bo)
    return outp[:_N]


# speedup vs baseline: 1.1816x; 1.1816x over previous
import functools

import jax
import jax.numpy as jnp
from jax import lax
from jax.experimental import pallas as pl
from jax.experimental.pallas import tpu as pltpu
from jax.experimental.pallas import tpu_sc as plsc

_N = 10000
_D = 128
_N_PAD = 10240
_CH = 64
_NSUB = 16
_NWORK = 32
_KCH = 158
_KCH0 = 192
_KCH1 = 2 * _KCH - _KCH0
_E_PAD = _NWORK * _KCH * _CH
_E_ALLOC = _E_PAD + 2 * _CH
_ROWS_PER_TILE = _N_PAD // _NSUB
_SLOPE = 0.2


_NDEN = _N_PAD // _D


def _edge_body(xl_hbm, xr_hbm, src_hbm, dst_hbm, a_hbm, msg_hbm, den_hbm,
               srcbuf, dstbuf, dscat, glbuf, grbuf, dloc, idxiota, wtmp, abuf,
               sacc, sden, semg0, semg1, semi0, semi1, sems0, sems1):
    c = lax.axis_index("c")
    s = lax.axis_index("s")
    wid = c * _NSUB + s
    semg = (semg0, semg1)
    semi = (semi0, semi1)
    sems = (sems0, sems1)

    pltpu.sync_copy(a_hbm, abuf)

    lanes = lax.iota(jnp.int32, 16)
    zero16 = jnp.zeros((16,), jnp.float32)

    def _zrow(i, carry):
        for j in range(_D // 16):
            glbuf[0, i, pl.ds(j * 16, 16)] = zero16
        return carry

    lax.fori_loop(0, _CH, _zrow, 0)

    def _zden(i, carry):
        for j in range(_D // 16):
            dloc[i, pl.ds(j * 16, 16)] = zero16
        return carry

    lax.fori_loop(0, _NDEN, _zden, 0)

    def _ziota(i, carry):
        idxiota[pl.ds(i * 16, 16)] = i * 16 + lanes
        return carry

    lax.fori_loop(0, _NDEN // 16, _ziota, 0)

    for k in range(_ROWS_PER_TILE // _CH):
        pltpu.sync_copy(glbuf.at[0],
                        sacc.at[pl.ds(s * _ROWS_PER_TILE + k * _CH, _CH)])

    @pl.when(s == 0)
    def _init_sden():
        pltpu.sync_copy(glbuf.at[0], sden.at[pl.ds(0, _CH)])
        pltpu.sync_copy(glbuf.at[0, pl.ds(0, _NDEN - _CH)],
                        sden.at[pl.ds(_CH, _NDEN - _CH)])

    plsc.subcore_barrier()

    av = [abuf[pl.ds(jb * 16, 16)] for jb in range(_D // 16)]
    lane0 = lanes == 0
    lanesm1 = jnp.maximum(lanes - 1, 0)
    lanesp1 = jnp.minimum(lanes + 1, 15)
    ebase = jnp.where(c == 0, s * _KCH0,
                      _NSUB * _KCH0 + s * _KCH1) * _CH
    nch = jnp.where(c == 0, _KCH0, _KCH1)

    def _idx_issue(slot, k):
        base = ebase + k * _CH
        pltpu.async_copy(src_hbm.at[pl.ds(base, _CH)], srcbuf.at[slot],
                         semi[slot])
        pltpu.async_copy(dst_hbm.at[pl.ds(base, _CH)], dstbuf.at[slot],
                         semi[slot])

    def _idx_drain(slot):
        pltpu.make_async_copy(src_hbm.at[pl.ds(0, _CH)], srcbuf.at[slot],
                              semi[slot]).wait()
        pltpu.make_async_copy(dst_hbm.at[pl.ds(0, _CH)], dstbuf.at[slot],
                              semi[slot]).wait()

    def _gather_issue(slot):
        pltpu.async_copy(xl_hbm.at[srcbuf.at[slot]], glbuf.at[slot],
                         semg[slot])
        pltpu.async_copy(xr_hbm.at[dstbuf.at[slot]], grbuf.at[slot],
                         semg[slot])

    def _gather_drain(slot):
        pltpu.make_async_copy(xl_hbm.at[srcbuf.at[slot]], glbuf.at[slot],
                              semg[slot]).wait()
        pltpu.make_async_copy(xr_hbm.at[dstbuf.at[slot]], grbuf.at[slot],
                              semg[slot]).wait()

    def _msg_drain(slot):
        pltpu.make_async_copy(grbuf.at[slot], sacc.at[dscat.at[slot]],
                              sems[slot]).wait()

    def _chunk_step(k, sl, other, first=False):
        _idx_drain(other)
        if not first:
            _msg_drain(other)
        _gather_issue(other)
        _gather_drain(sl)

        @plsc.parallel_loop(0, _CH, unroll=2)
        def _edge(e):
            acc = jnp.zeros((16,), jnp.float32)
            gls = []
            for jb in range(_D // 16):
                gl = glbuf[sl, e, pl.ds(jb * 16, 16)]
                gr = grbuf[sl, e, pl.ds(jb * 16, 16)]
                gls.append(gl)
                m = gl + gr
                lr = jnp.maximum(m, _SLOPE * m)
                acc = acc + av[jb] * lr
            logit = jnp.sum(acc)
            w = jnp.exp(jnp.full((16,), logit, jnp.float32))
            for jb in range(_D // 16):
                grbuf[sl, e, pl.ds(jb * 16, 16)] = gls[jb] * w
            plsc.store_scatter(wtmp, [jnp.full((16,), e, jnp.int32)], w,
                               mask=lane0)

        @plsc.parallel_loop(0, _CH // 16, unroll=2)
        def _dcopy(g):
            dscat[sl, pl.ds(g * 16, 16)] = dstbuf[sl, pl.ds(g * 16, 16)]

        pltpu.async_copy(grbuf.at[sl], sacc.at[dscat.at[sl]], sems[sl],
                         add=True)

        @plsc.parallel_loop(0, _CH // 16, unroll=2)
        def _dgroup(g):
            dstv = dscat[sl, pl.ds(g * 16, 16)]
            w16 = wtmp[pl.ds(g * 16, 16)]
            ks, vs = plsc.sort_key_val(dstv, w16)
            ksh = ks.at[lanesm1].get(mode="promise_in_bounds")
            b = (lanes == 0) | (ks != ksh)
            st = plsc.cummax(jnp.where(b, lanes, 0))
            cs = plsc.cumsum(vs)
            csm = cs.at[jnp.maximum(st - 1, 0)].get(mode="promise_in_bounds")
            seg = cs - jnp.where(st == 0, 0.0, csm)
            ksn = ks.at[lanesp1].get(mode="promise_in_bounds")
            end = (lanes == 15) | (ks != ksn)
            rowi = lax.shift_right_logical(ks, 7)
            coli = lax.bitwise_and(ks, _D - 1)
            plsc.addupdate_scatter(dloc, [rowi, coli], seg, mask=end)

        _idx_issue(sl, k + 2)

    pltpu.sync_copy(src_hbm.at[pl.ds(ebase, _CH)], srcbuf.at[0])
    pltpu.sync_copy(dst_hbm.at[pl.ds(ebase, _CH)], dstbuf.at[0])
    _gather_issue(0)
    _idx_issue(1, 1)

    _chunk_step(0, 0, 1, first=True)
    _chunk_step(1, 1, 0)

    def _pair(kk, carry):
        k = kk * 2
        _chunk_step(k, 0, 1)
        _chunk_step(k + 1, 1, 0)
        return carry

    lax.fori_loop(1, nch // 2, _pair, 0)

    _gather_drain(0)
    _idx_drain(1)
    _msg_drain(1)

    pltpu.sync_copy(dloc, sden.at[idxiota], add=True)

    plsc.subcore_barrier()
    rb = s * _ROWS_PER_TILE
    pltpu.sync_copy(sacc.at[pl.ds(rb, _ROWS_PER_TILE)],
                    msg_hbm.at[c, pl.ds(rb, _ROWS_PER_TILE)])

    @pl.when(s < 10)
    def _den_out():
        pltpu.sync_copy(sden.at[pl.ds(s * 8, 8)],
                        den_hbm.at[c, pl.ds(s * 8, 8)])


def _edge_call(xl, xr, srcp, dstp, a):
    mesh = plsc.VectorSubcoreMesh(core_axis_name="c", subcore_axis_name="s")
    fn = pl.kernel(
        _edge_body,
        mesh=mesh,
        compiler_params=pltpu.CompilerParams(needs_layout_passes=False),
        out_type=[
            jax.ShapeDtypeStruct((2, _N_PAD, _D), jnp.float32),
            jax.ShapeDtypeStruct((2, _NDEN, _D), jnp.float32),
        ],
        scratch_types=[
            pltpu.VMEM((2, _CH), jnp.int32),
            pltpu.VMEM((2, _CH), jnp.int32),
            pltpu.VMEM((2, _CH), jnp.int32),
            pltpu.VMEM((2, _CH, _D), jnp.float32),
            pltpu.VMEM((2, _CH, _D), jnp.float32),
            pltpu.VMEM((_NDEN, _D), jnp.float32),
            pltpu.VMEM((_NDEN,), jnp.int32),
            pltpu.VMEM((_CH,), jnp.float32),
            pltpu.VMEM((_D,), jnp.float32),
            pltpu.VMEM_SHARED((_N_PAD, _D), jnp.float32),
            pltpu.VMEM_SHARED((_NDEN, _D), jnp.float32),
            pltpu.SemaphoreType.DMA,
            pltpu.SemaphoreType.DMA,
            pltpu.SemaphoreType.DMA,
            pltpu.SemaphoreType.DMA,
            pltpu.SemaphoreType.DMA,
            pltpu.SemaphoreType.DMA,
        ],
    )
    return fn(xl, xr, srcp, dstp, a)


_ROWS_BLK = 1024


def _prologue_body(x_ref, wl_ref, bl_ref, wr_ref, br_ref, xl_ref, xr_ref):
    x = x_ref[...]
    xl_ref[...] = jnp.dot(x, wl_ref[...], preferred_element_type=jnp.float32) + bl_ref[...]
    xr_ref[...] = jnp.dot(x, wr_ref[...], preferred_element_type=jnp.float32) + br_ref[...]


def _combine_h(xl, xr, am0, am1, ad0, ad1, a_row, bias_row):
    ssum = xl + xr
    lr = jnp.maximum(ssum, _SLOPE * ssum)
    cself = jnp.sum(lr * a_row, axis=1, keepdims=True)
    scv = jnp.exp(cself)
    num = scv * xl + am0 + am1
    den = scv + ad0[:, 0:1] + ad1[:, 0:1] + 1e-16
    return jnp.maximum(num / den + bias_row, 0.0)


def _combine_prologue_body(xl_ref, xr_ref, am0_ref, am1_ref, ad0_ref, ad1_ref,
                           a_ref, bias_ref, wl_ref, bl_ref, wr_ref, br_ref,
                           ol_ref, or_ref):
    h = _combine_h(xl_ref[...], xr_ref[...], am0_ref[...], am1_ref[...],
                   ad0_ref[...], ad1_ref[...], a_ref[...], bias_ref[...])
    ol_ref[...] = jnp.dot(h, wl_ref[...], preferred_element_type=jnp.float32) + bl_ref[...]
    or_ref[...] = jnp.dot(h, wr_ref[...], preferred_element_type=jnp.float32) + br_ref[...]


def _final_body(xl_ref, xr_ref, am0_ref, am1_ref, ad0_ref, ad1_ref,
                a_ref, bias_ref, wo_ref, bo_ref, out_ref):
    h = _combine_h(xl_ref[...], xr_ref[...], am0_ref[...], am1_ref[...],
                   ad0_ref[...], ad1_ref[...], a_ref[...], bias_ref[...])
    z = jnp.dot(h, wo_ref[...], preferred_element_type=jnp.float32) + bo_ref[...]
    m = jnp.max(z, axis=1, keepdims=True)
    ez = jnp.exp(z - m)
    out_ref[...] = (z - m) - jnp.log(jnp.sum(ez, axis=1, keepdims=True))


def _row_spec():
    return pl.BlockSpec((_ROWS_BLK, _D), lambda i: (i, 0))


def _den_spec():
    return pl.BlockSpec((_ROWS_BLK, 16), lambda i: (i, 0))


def _w_spec():
    return pl.BlockSpec((_D, _D), lambda i: (0, 0))


def _b_spec():
    return pl.BlockSpec((1, _D), lambda i: (0, 0))


def _tc_prologue(x, wl, bl, wr, br):
    grid = (_N_PAD // _ROWS_BLK,)
    return pl.pallas_call(
        _prologue_body,
        grid=grid,
        in_specs=[_row_spec(), _w_spec(), _b_spec(), _w_spec(), _b_spec()],
        out_specs=[_row_spec(), _row_spec()],
        out_shape=[jax.ShapeDtypeStruct((_N_PAD, _D), jnp.float32)] * 2,
    )(x, wl, bl.reshape(1, _D), wr, br.reshape(1, _D))


def _split_acc(msg, den):
    denv = den.reshape(2, _N_PAD)
    ad0 = jnp.broadcast_to(denv[0][:, None], (_N_PAD, 16))
    ad1 = jnp.broadcast_to(denv[1][:, None], (_N_PAD, 16))
    return msg[0], msg[1], ad0, ad1


def _tc_combine_prologue(xl, xr, acc, a, bias, wl, bl, wr, br):
    grid = (_N_PAD // _ROWS_BLK,)
    am0, am1, ad0, ad1 = _split_acc(*acc)
    return pl.pallas_call(
        _combine_prologue_body,
        grid=grid,
        in_specs=[_row_spec(), _row_spec(), _row_spec(), _row_spec(),
                  _den_spec(), _den_spec(), _b_spec(), _b_spec(),
                  _w_spec(), _b_spec(), _w_spec(), _b_spec()],
        out_specs=[_row_spec(), _row_spec()],
        out_shape=[jax.ShapeDtypeStruct((_N_PAD, _D), jnp.float32)] * 2,
    )(xl, xr, am0, am1, ad0, ad1, a.reshape(1, _D), bias.reshape(1, _D),
      wl, bl.reshape(1, _D), wr, br.reshape(1, _D))


def _tc_final(xl, xr, acc, a, bias, wo, bo):
    grid = (_N_PAD // _ROWS_BLK,)
    am0, am1, ad0, ad1 = _split_acc(*acc)
    return pl.pallas_call(
        _final_body,
        grid=grid,
        in_specs=[_row_spec(), _row_spec(), _row_spec(), _row_spec(),
                  _den_spec(), _den_spec(), _b_spec(), _b_spec(),
                  _w_spec(), _b_spec()],
        out_specs=_row_spec(),
        out_shape=jax.ShapeDtypeStruct((_N_PAD, _D), jnp.float32),
    )(xl, xr, am0, am1, ad0, ad1, a.reshape(1, _D), bias.reshape(1, _D),
      wo, bo.reshape(1, _D))


def kernel(x, edge_index, W1l, b1l, W1r, b1r, a1, c1, W2l, b2l, W2r, b2r,
           a2, c2, Wo, bo):
    xp = jnp.pad(x, ((0, _N_PAD - _N), (0, 0)))
    src = edge_index[0].astype(jnp.int32)
    dst = edge_index[1].astype(jnp.int32)
    pad_e = _E_ALLOC - src.shape[0]
    srcp = jnp.concatenate([src, jnp.zeros((pad_e,), jnp.int32)])
    dstp = jnp.concatenate([dst, jnp.full((pad_e,), _N, jnp.int32)])

    xl1, xr1 = _tc_prologue(xp, W1l, b1l, W1r, b1r)
    acc1 = _edge_call(xl1, xr1, srcp, dstp, a1)
    xl2, xr2 = _tc_combine_prologue(xl1, xr1, acc1, a1, c1, W2l, b2l, W2r, b2r)
    acc2 = _edge_call(xl2, xr2, srcp, dstp, a2)
    outp = _tc_final(xl2, xr2, acc2, a2, c2, Wo, bo)
    return outp[:_N]


# speedup vs baseline: 1.2056x; 1.0203x over previous
import functools

import jax
import jax.numpy as jnp
from jax import lax
from jax.experimental import pallas as pl
from jax.experimental.pallas import tpu as pltpu
from jax.experimental.pallas import tpu_sc as plsc

_N = 10000
_D = 128
_N_PAD = 10240
_CH = 64
_NSUB = 16
_NWORK = 32
_KCH = 158
_KCH0 = 200
_KCH1 = 2 * _KCH - _KCH0
_E_PAD = _NWORK * _KCH * _CH
_E_ALLOC = _E_PAD + 2 * _CH
_ROWS_PER_TILE = _N_PAD // _NSUB
_SLOPE = 0.2


_NDEN = _N_PAD // _D


def _edge_body(xl_hbm, xr_hbm, src_hbm, dst_hbm, a_hbm, msg_hbm, den_hbm,
               srcbuf, dstbuf, dscat, glbuf, grbuf, dloc, idxiota, wtmp, abuf,
               sacc, sden, semg0, semg1, semi0, semi1, sems0, sems1):
    c = lax.axis_index("c")
    s = lax.axis_index("s")
    wid = c * _NSUB + s
    semg = (semg0, semg1)
    semi = (semi0, semi1)
    sems = (sems0, sems1)

    pltpu.sync_copy(a_hbm, abuf)

    lanes = lax.iota(jnp.int32, 16)
    zero16 = jnp.zeros((16,), jnp.float32)

    def _zrow(i, carry):
        for j in range(_D // 16):
            glbuf[0, i, pl.ds(j * 16, 16)] = zero16
        return carry

    lax.fori_loop(0, _CH, _zrow, 0)

    def _zden(i, carry):
        for j in range(_D // 16):
            dloc[i, pl.ds(j * 16, 16)] = zero16
        return carry

    lax.fori_loop(0, _NDEN, _zden, 0)

    def _ziota(i, carry):
        idxiota[pl.ds(i * 16, 16)] = i * 16 + lanes
        return carry

    lax.fori_loop(0, _NDEN // 16, _ziota, 0)

    for k in range(_ROWS_PER_TILE // _CH):
        pltpu.sync_copy(glbuf.at[0],
                        sacc.at[pl.ds(s * _ROWS_PER_TILE + k * _CH, _CH)])

    @pl.when(s == 0)
    def _init_sden():
        pltpu.sync_copy(glbuf.at[0], sden.at[pl.ds(0, _CH)])
        pltpu.sync_copy(glbuf.at[0, pl.ds(0, _NDEN - _CH)],
                        sden.at[pl.ds(_CH, _NDEN - _CH)])

    plsc.subcore_barrier()

    av = [abuf[pl.ds(jb * 16, 16)] for jb in range(_D // 16)]
    lane0 = lanes == 0
    lanesm1 = jnp.maximum(lanes - 1, 0)
    lanesp1 = jnp.minimum(lanes + 1, 15)
    ebase = jnp.where(c == 0, s * _KCH0,
                      _NSUB * _KCH0 + s * _KCH1) * _CH
    nch = jnp.where(c == 0, _KCH0, _KCH1)

    def _idx_issue(slot, k):
        base = ebase + k * _CH
        pltpu.async_copy(src_hbm.at[pl.ds(base, _CH)], srcbuf.at[slot],
                         semi[slot])
        pltpu.async_copy(dst_hbm.at[pl.ds(base, _CH)], dstbuf.at[slot],
                         semi[slot])

    def _idx_drain(slot):
        pltpu.make_async_copy(src_hbm.at[pl.ds(0, _CH)], srcbuf.at[slot],
                              semi[slot]).wait()
        pltpu.make_async_copy(dst_hbm.at[pl.ds(0, _CH)], dstbuf.at[slot],
                              semi[slot]).wait()

    def _gather_issue(slot):
        pltpu.async_copy(xl_hbm.at[srcbuf.at[slot]], glbuf.at[slot],
                         semg[slot])
        pltpu.async_copy(xr_hbm.at[dstbuf.at[slot]], grbuf.at[slot],
                         semg[slot])

    def _gather_drain(slot):
        pltpu.make_async_copy(xl_hbm.at[srcbuf.at[slot]], glbuf.at[slot],
                              semg[slot]).wait()
        pltpu.make_async_copy(xr_hbm.at[dstbuf.at[slot]], grbuf.at[slot],
                              semg[slot]).wait()

    def _msg_drain(slot):
        pltpu.make_async_copy(grbuf.at[slot], sacc.at[dscat.at[slot]],
                              sems[slot]).wait()

    def _chunk_step(k, sl, other, first=False):
        _idx_drain(other)
        if not first:
            _msg_drain(other)
        _gather_issue(other)
        _gather_drain(sl)

        @plsc.parallel_loop(0, _CH, unroll=2)
        def _edge(e):
            acc = jnp.zeros((16,), jnp.float32)
            gls = []
            for jb in range(_D // 16):
                gl = glbuf[sl, e, pl.ds(jb * 16, 16)]
                gr = grbuf[sl, e, pl.ds(jb * 16, 16)]
                gls.append(gl)
                m = gl + gr
                lr = jnp.maximum(m, _SLOPE * m)
                acc = acc + av[jb] * lr
            logit = jnp.sum(acc)
            w = jnp.exp(jnp.full((16,), logit, jnp.float32))
            for jb in range(_D // 16):
                grbuf[sl, e, pl.ds(jb * 16, 16)] = gls[jb] * w
            plsc.store_scatter(wtmp, [jnp.full((16,), e, jnp.int32)], w,
                               mask=lane0)

        @plsc.parallel_loop(0, _CH // 16, unroll=2)
        def _dcopy(g):
            dscat[sl, pl.ds(g * 16, 16)] = dstbuf[sl, pl.ds(g * 16, 16)]

        pltpu.async_copy(grbuf.at[sl], sacc.at[dscat.at[sl]], sems[sl],
                         add=True)

        @plsc.parallel_loop(0, _CH // 16, unroll=2)
        def _dgroup(g):
            dstv = dscat[sl, pl.ds(g * 16, 16)]
            w16 = wtmp[pl.ds(g * 16, 16)]
            ks, vs = plsc.sort_key_val(dstv, w16)
            ksh = ks.at[lanesm1].get(mode="promise_in_bounds")
            b = (lanes == 0) | (ks != ksh)
            st = plsc.cummax(jnp.where(b, lanes, 0))
            cs = plsc.cumsum(vs)
            csm = cs.at[jnp.maximum(st - 1, 0)].get(mode="promise_in_bounds")
            seg = cs - jnp.where(st == 0, 0.0, csm)
            ksn = ks.at[lanesp1].get(mode="promise_in_bounds")
            end = (lanes == 15) | (ks != ksn)
            rowi = lax.shift_right_logical(ks, 7)
            coli = lax.bitwise_and(ks, _D - 1)
            plsc.addupdate_scatter(dloc, [rowi, coli], seg, mask=end)

        _idx_issue(sl, k + 2)

    pltpu.sync_copy(src_hbm.at[pl.ds(ebase, _CH)], srcbuf.at[0])
    pltpu.sync_copy(dst_hbm.at[pl.ds(ebase, _CH)], dstbuf.at[0])
    _gather_issue(0)
    _idx_issue(1, 1)

    _chunk_step(0, 0, 1, first=True)
    _chunk_step(1, 1, 0)

    def _pair(kk, carry):
        k = kk * 2
        _chunk_step(k, 0, 1)
        _chunk_step(k + 1, 1, 0)
        return carry

    lax.fori_loop(1, nch // 2, _pair, 0)

    _gather_drain(0)
    _idx_drain(1)
    _msg_drain(1)

    pltpu.sync_copy(dloc, sden.at[idxiota], add=True)

    plsc.subcore_barrier()
    rb = s * _ROWS_PER_TILE
    pltpu.sync_copy(sacc.at[pl.ds(rb, _ROWS_PER_TILE)],
                    msg_hbm.at[c, pl.ds(rb, _ROWS_PER_TILE)])

    @pl.when(s < 10)
    def _den_out():
        pltpu.sync_copy(sden.at[pl.ds(s * 8, 8)],
                        den_hbm.at[c, pl.ds(s * 8, 8)])


def _edge_call(xl, xr, srcp, dstp, a):
    mesh = plsc.VectorSubcoreMesh(core_axis_name="c", subcore_axis_name="s")
    fn = pl.kernel(
        _edge_body,
        mesh=mesh,
        compiler_params=pltpu.CompilerParams(needs_layout_passes=False),
        out_type=[
            jax.ShapeDtypeStruct((2, _N_PAD, _D), jnp.float32),
            jax.ShapeDtypeStruct((2, _NDEN, _D), jnp.float32),
        ],
        scratch_types=[
            pltpu.VMEM((2, _CH), jnp.int32),
            pltpu.VMEM((2, _CH), jnp.int32),
            pltpu.VMEM((2, _CH), jnp.int32),
            pltpu.VMEM((2, _CH, _D), jnp.float32),
            pltpu.VMEM((2, _CH, _D), jnp.float32),
            pltpu.VMEM((_NDEN, _D), jnp.float32),
            pltpu.VMEM((_NDEN,), jnp.int32),
            pltpu.VMEM((_CH,), jnp.float32),
            pltpu.VMEM((_D,), jnp.float32),
            pltpu.VMEM_SHARED((_N_PAD, _D), jnp.float32),
            pltpu.VMEM_SHARED((_NDEN, _D), jnp.float32),
            pltpu.SemaphoreType.DMA,
            pltpu.SemaphoreType.DMA,
            pltpu.SemaphoreType.DMA,
            pltpu.SemaphoreType.DMA,
            pltpu.SemaphoreType.DMA,
            pltpu.SemaphoreType.DMA,
        ],
    )
    return fn(xl, xr, srcp, dstp, a)


_ROWS_BLK = 1024


def _prologue_body(x_ref, wl_ref, bl_ref, wr_ref, br_ref, xl_ref, xr_ref):
    x = x_ref[...]
    xl_ref[...] = jnp.dot(x, wl_ref[...], preferred_element_type=jnp.float32) + bl_ref[...]
    xr_ref[...] = jnp.dot(x, wr_ref[...], preferred_element_type=jnp.float32) + br_ref[...]


def _combine_h(xl, xr, am0, am1, ad0, ad1, a_row, bias_row):
    ssum = xl + xr
    lr = jnp.maximum(ssum, _SLOPE * ssum)
    cself = jnp.sum(lr * a_row, axis=1, keepdims=True)
    scv = jnp.exp(cself)
    num = scv * xl + am0 + am1
    den = scv + ad0[:, 0:1] + ad1[:, 0:1] + 1e-16
    return jnp.maximum(num / den + bias_row, 0.0)


def _combine_prologue_body(xl_ref, xr_ref, am0_ref, am1_ref, ad0_ref, ad1_ref,
                           a_ref, bias_ref, wl_ref, bl_ref, wr_ref, br_ref,
                           ol_ref, or_ref):
    h = _combine_h(xl_ref[...], xr_ref[...], am0_ref[...], am1_ref[...],
                   ad0_ref[...], ad1_ref[...], a_ref[...], bias_ref[...])
    ol_ref[...] = jnp.dot(h, wl_ref[...], preferred_element_type=jnp.float32) + bl_ref[...]
    or_ref[...] = jnp.dot(h, wr_ref[...], preferred_element_type=jnp.float32) + br_ref[...]


def _final_body(xl_ref, xr_ref, am0_ref, am1_ref, ad0_ref, ad1_ref,
                a_ref, bias_ref, wo_ref, bo_ref, out_ref):
    h = _combine_h(xl_ref[...], xr_ref[...], am0_ref[...], am1_ref[...],
                   ad0_ref[...], ad1_ref[...], a_ref[...], bias_ref[...])
    z = jnp.dot(h, wo_ref[...], preferred_element_type=jnp.float32) + bo_ref[...]
    m = jnp.max(z, axis=1, keepdims=True)
    ez = jnp.exp(z - m)
    out_ref[...] = (z - m) - jnp.log(jnp.sum(ez, axis=1, keepdims=True))


def _row_spec():
    return pl.BlockSpec((_ROWS_BLK, _D), lambda i: (i, 0))


def _den_spec():
    return pl.BlockSpec((_ROWS_BLK, 16), lambda i: (i, 0))


def _w_spec():
    return pl.BlockSpec((_D, _D), lambda i: (0, 0))


def _b_spec():
    return pl.BlockSpec((1, _D), lambda i: (0, 0))


def _tc_prologue(x, wl, bl, wr, br):
    grid = (_N_PAD // _ROWS_BLK,)
    return pl.pallas_call(
        _prologue_body,
        grid=grid,
        in_specs=[_row_spec(), _w_spec(), _b_spec(), _w_spec(), _b_spec()],
        out_specs=[_row_spec(), _row_spec()],
        out_shape=[jax.ShapeDtypeStruct((_N_PAD, _D), jnp.float32)] * 2,
    )(x, wl, bl.reshape(1, _D), wr, br.reshape(1, _D))


def _split_acc(msg, den):
    denv = den.reshape(2, _N_PAD)
    ad0 = jnp.broadcast_to(denv[0][:, None], (_N_PAD, 16))
    ad1 = jnp.broadcast_to(denv[1][:, None], (_N_PAD, 16))
    return msg[0], msg[1], ad0, ad1


def _tc_combine_prologue(xl, xr, acc, a, bias, wl, bl, wr, br):
    grid = (_N_PAD // _ROWS_BLK,)
    am0, am1, ad0, ad1 = _split_acc(*acc)
    return pl.pallas_call(
        _combine_prologue_body,
        grid=grid,
        in_specs=[_row_spec(), _row_spec(), _row_spec(), _row_spec(),
                  _den_spec(), _den_spec(), _b_spec(), _b_spec(),
                  _w_spec(), _b_spec(), _w_spec(), _b_spec()],
        out_specs=[_row_spec(), _row_spec()],
        out_shape=[jax.ShapeDtypeStruct((_N_PAD, _D), jnp.float32)] * 2,
    )(xl, xr, am0, am1, ad0, ad1, a.reshape(1, _D), bias.reshape(1, _D),
      wl, bl.reshape(1, _D), wr, br.reshape(1, _D))


def _tc_final(xl, xr, acc, a, bias, wo, bo):
    grid = (_N_PAD // _ROWS_BLK,)
    am0, am1, ad0, ad1 = _split_acc(*acc)
    return pl.pallas_call(
        _final_body,
        grid=grid,
        in_specs=[_row_spec(), _row_spec(), _row_spec(), _row_spec(),
                  _den_spec(), _den_spec(), _b_spec(), _b_spec(),
                  _w_spec(), _b_spec()],
        out_specs=_row_spec(),
        out_shape=jax.ShapeDtypeStruct((_N_PAD, _D), jnp.float32),
    )(xl, xr, am0, am1, ad0, ad1, a.reshape(1, _D), bias.reshape(1, _D),
      wo, bo.reshape(1, _D))


def kernel(x, edge_index, W1l, b1l, W1r, b1r, a1, c1, W2l, b2l, W2r, b2r,
           a2, c2, Wo, bo):
    xp = jnp.pad(x, ((0, _N_PAD - _N), (0, 0)))
    src = edge_index[0].astype(jnp.int32)
    dst = edge_index[1].astype(jnp.int32)
    pad_e = _E_ALLOC - src.shape[0]
    srcp = jnp.concatenate([src, jnp.zeros((pad_e,), jnp.int32)])
    dstp = jnp.concatenate([dst, jnp.full((pad_e,), _N, jnp.int32)])

    xl1, xr1 = _tc_prologue(xp, W1l, b1l, W1r, b1r)
    acc1 = _edge_call(xl1, xr1, srcp, dstp, a1)
    xl2, xr2 = _tc_combine_prologue(xl1, xr1, acc1, a1, c1, W2l, b2l, W2r, b2r)
    acc2 = _edge_call(xl2, xr2, srcp, dstp, a2)
    outp = _tc_final(xl2, xr2, acc2, a2, c2, Wo, bo)
    return outp[:_N]
